# Initial kernel scaffold; baseline (speedup 1.0000x reference)
import jax
import jax.numpy as jnp
from jax.experimental import pallas as pl


def kernel(x, y, edge_index, gcn_W0, gcn_b0, gcn_W1, gcn_b1, lab_W, lab_b, fus_W, fus_b):
    raise NotImplementedError("write your pallas kernel here")


# speedup vs baseline: 8.1935x; 8.1935x over previous
import functools

import jax
import jax.numpy as jnp
from jax import lax
from jax.experimental import pallas as pl
from jax.experimental.pallas import tpu as pltpu
from jax.experimental.pallas import tpu_sc as plsc

NC = 2
NS = 16
NW = NC * NS
CH = 80

_F32 = jnp.float32


@functools.cache
def _make_deg(N, E):
    epw = E // NW
    nchunk = epw // CH
    zch = 2000
    nzw = N // zch
    mesh = plsc.VectorSubcoreMesh(core_axis_name="c", subcore_axis_name="s")

    @functools.partial(
        pl.kernel,
        out_type=jax.ShapeDtypeStruct((NC * N,), _F32),
        mesh=mesh,
        scratch_types=[
            pltpu.VMEM((CH,), jnp.int32),
            pltpu.VMEM((CH,), _F32),
            pltpu.VMEM((zch,), _F32),
            pltpu.VMEM_SHARED((N,), _F32),
        ],
    )
    def deg_kernel(dst_hbm, out_hbm, didx, ones, zbuf, acc):
        cid = lax.axis_index("c")
        sid = lax.axis_index("s")
        wid = sid * NC + cid
        base = wid * epw
        for j in range(CH // 16):
            ones[pl.ds(j * 16, 16)] = jnp.full((16,), 1.0, _F32)

        @pl.when(sid < nzw)
        def _():
            def zstep(j, carry):
                zbuf[pl.ds(j * 16, 16)] = jnp.zeros((16,), _F32)
                return carry
            lax.fori_loop(0, zch // 16, zstep, 0)
            pltpu.sync_copy(zbuf, acc.at[pl.ds(sid * zch, zch)])

        plsc.subcore_barrier()

        def step(i, carry):
            pltpu.sync_copy(dst_hbm.at[pl.ds(base + i * CH, CH)], didx)
            pltpu.sync_copy(ones, acc.at[didx], add=True)
            return carry

        lax.fori_loop(0, nchunk, step, 0)
        plsc.subcore_barrier()

        @pl.when(sid < nzw)
        def _():
            pltpu.sync_copy(acc.at[pl.ds(sid * zch, zch)], zbuf)
            pltpu.sync_copy(zbuf, out_hbm.at[pl.ds(cid * N + sid * zch, zch)])

    return deg_kernel


@functools.cache
def _make_segsum(N, D, E):
    epw = E // NW
    nchunk = epw // CH
    zch = 1000
    nzw = N // zch
    mesh = plsc.VectorSubcoreMesh(core_axis_name="c", subcore_axis_name="s")

    @functools.partial(
        pl.kernel,
        out_type=jax.ShapeDtypeStruct((NC, N, D), _F32),
        mesh=mesh,
        compiler_params=pltpu.CompilerParams(use_tc_tiling_on_sc=False),
        scratch_types=[
            pltpu.VMEM((CH,), jnp.int32),
            pltpu.VMEM((CH,), jnp.int32),
            pltpu.VMEM((CH, D), _F32),
            pltpu.SemaphoreType.DMA,
            pltpu.VMEM_SHARED((N, D), _F32),
        ],
    )
    def seg_kernel(hs_hbm, src_hbm, dst_hbm, zeros_hbm, out_hbm,
                   sidx, didx, rows, sem, acc):
        cid = lax.axis_index("c")
        sid = lax.axis_index("s")
        wid = sid * NC + cid
        base = wid * epw

        @pl.when(sid < nzw)
        def _():
            pltpu.sync_copy(zeros_hbm.at[pl.ds(sid * zch, zch)],
                            acc.at[pl.ds(sid * zch, zch)])

        plsc.subcore_barrier()

        def step(i, carry):
            off = base + i * CH
            pltpu.sync_copy(src_hbm.at[pl.ds(off, CH)], sidx)
            pltpu.sync_copy(dst_hbm.at[pl.ds(off, CH)], didx)
            pltpu.async_copy(hs_hbm.at[sidx], rows, sem).wait()
            pltpu.sync_copy(rows, acc.at[didx], add=True)
            return carry

        lax.fori_loop(0, nchunk, step, 0)
        plsc.subcore_barrier()

        @pl.when(sid < nzw)
        def _():
            pltpu.sync_copy(acc.at[pl.ds(sid * zch, zch)],
                            out_hbm.at[cid, pl.ds(sid * zch, zch)])

    return seg_kernel


def _prep_body(p_ref, dis_ref, inv_ref):
    deg = p_ref[:, 0:1] + p_ref[:, 1:2] + 1.0
    dis_ref[:] = lax.rsqrt(deg)
    inv_ref[:] = 1.0 / deg


def _prep(deg_parts):
    n = deg_parts.shape[0]
    return pl.pallas_call(
        _prep_body,
        out_shape=(jax.ShapeDtypeStruct((n, 1), _F32),
                   jax.ShapeDtypeStruct((n, 1), _F32)),
    )(deg_parts)


def _mm_scale_body(x_ref, w_ref, dis_ref, hs_ref):
    h = jnp.dot(x_ref[:], w_ref[:], preferred_element_type=_F32)
    hs_ref[:] = h * dis_ref[:]


def _mm_scale(x, w, dis, bn=2000):
    n, k = x.shape
    m = w.shape[1]
    return pl.pallas_call(
        _mm_scale_body,
        grid=(n // bn,),
        in_specs=[
            pl.BlockSpec((bn, k), lambda i: (i, 0)),
            pl.BlockSpec((k, m), lambda i: (0, 0)),
            pl.BlockSpec((bn, 1), lambda i: (i, 0)),
        ],
        out_specs=pl.BlockSpec((bn, m), lambda i: (i, 0)),
        out_shape=jax.ShapeDtypeStruct((n, m), _F32),
    )(x, w, dis)


def _combine_mm_body(relu, p0_ref, p1_ref, hs_ref, dis_ref, b_ref, w_ref,
                     out_ref):
    a = dis_ref[:] * (p0_ref[:] + p1_ref[:] + hs_ref[:]) + b_ref[:]
    if relu:
        a = jnp.maximum(a, 0.0)
    h = jnp.dot(a, w_ref[:], preferred_element_type=_F32)
    out_ref[:] = h * dis_ref[:]


def _combine_mm(p0, p1, hs, dis, b, w, relu, bn=2000):
    n, k = hs.shape
    m = w.shape[1]
    return pl.pallas_call(
        functools.partial(_combine_mm_body, relu),
        grid=(n // bn,),
        in_specs=[
            pl.BlockSpec((bn, k), lambda i: (i, 0)),
            pl.BlockSpec((bn, k), lambda i: (i, 0)),
            pl.BlockSpec((bn, k), lambda i: (i, 0)),
            pl.BlockSpec((bn, 1), lambda i: (i, 0)),
            pl.BlockSpec((1, k), lambda i: (0, 0)),
            pl.BlockSpec((k, m), lambda i: (0, 0)),
        ],
        out_specs=pl.BlockSpec((bn, m), lambda i: (i, 0)),
        out_shape=jax.ShapeDtypeStruct((n, m), _F32),
    )(p0, p1, hs, dis, b, w)


def _fuse_body(pg0_ref, pg1_ref, hsg_ref, bg_ref,
               pl0_ref, pl1_ref, hsl_ref, bl_ref,
               wg_ref, wl_ref, fb_ref, dis_ref, out_ref):
    cg = dis_ref[:] * (pg0_ref[:] + pg1_ref[:] + hsg_ref[:]) + bg_ref[:]
    cl = dis_ref[:] * (pl0_ref[:] + pl1_ref[:] + hsl_ref[:]) + bl_ref[:]
    z = (jnp.dot(cg, wg_ref[:], preferred_element_type=_F32)
         + jnp.dot(cl, wl_ref[:], preferred_element_type=_F32)
         + fb_ref[:])
    out_ref[:] = jax.nn.sigmoid(z)


def _fuse(pg0, pg1, hsg, bg, pl0, pl1, hsl, bl, wg, wl, fb, dis, bn=2000):
    n, kg = hsg.shape
    kl = hsl.shape[1]
    m = wg.shape[1]
    return pl.pallas_call(
        _fuse_body,
        grid=(n // bn,),
        in_specs=[
            pl.BlockSpec((bn, kg), lambda i: (i, 0)),
            pl.BlockSpec((bn, kg), lambda i: (i, 0)),
            pl.BlockSpec((bn, kg), lambda i: (i, 0)),
            pl.BlockSpec((1, kg), lambda i: (0, 0)),
            pl.BlockSpec((bn, kl), lambda i: (i, 0)),
            pl.BlockSpec((bn, kl), lambda i: (i, 0)),
            pl.BlockSpec((bn, kl), lambda i: (i, 0)),
            pl.BlockSpec((1, kl), lambda i: (0, 0)),
            pl.BlockSpec((kg, m), lambda i: (0, 0)),
            pl.BlockSpec((kl, m), lambda i: (0, 0)),
            pl.BlockSpec((1, m), lambda i: (0, 0)),
            pl.BlockSpec((bn, 1), lambda i: (i, 0)),
        ],
        out_specs=pl.BlockSpec((bn, m), lambda i: (i, 0)),
        out_shape=jax.ShapeDtypeStruct((n, m), _F32),
    )(pg0, pg1, hsg, bg, pl0, pl1, hsl, bl, wg, wl, fb, dis)


def kernel(x, y, edge_index, gcn_W0, gcn_b0, gcn_W1, gcn_b1,
           lab_W, lab_b, fus_W, fus_b):
    n, d_in = x.shape
    e = edge_index.shape[1]
    d_hid = gcn_W0.shape[1]
    d_out = lab_W.shape[2]
    n_lab = lab_W.shape[0]

    src = edge_index[0]
    dst = edge_index[1]
    zeros_hid = jnp.zeros((n, d_hid), _F32)
    zeros_out = jnp.zeros((n, d_out), _F32)

    deg_parts = _make_deg(n, e)(dst).reshape(NC, n)
    dis, _inv = _prep(deg_parts.T)

    seg_hid = _make_segsum(n, d_hid, e)
    seg_out = _make_segsum(n, d_out, e)

    hs0 = _mm_scale(x, gcn_W0, dis)
    p = seg_hid(hs0, src, dst, zeros_hid)
    hs1 = _combine_mm(p[0], p[1], hs0, dis,
                      gcn_b0.reshape(1, -1), gcn_W1, relu=True)
    pg = seg_hid(hs1, src, dst, zeros_hid)

    ls = _mm_scale(y, lab_W[0], dis)
    for j in range(n_lab - 1):
        p = seg_out(ls, src, dst, zeros_out)
        ls = _combine_mm(p[0], p[1], ls, dis,
                         lab_b[j].reshape(1, -1), lab_W[j + 1], relu=True)
    pl_last = seg_out(ls, src, dst, zeros_out)

    return _fuse(pg[0], pg[1], hs1, gcn_b1.reshape(1, -1),
                 pl_last[0], pl_last[1], ls, lab_b[n_lab - 1].reshape(1, -1),
                 fus_W[:d_hid], fus_W[d_hid:], fus_b.reshape(1, -1), dis)
